# baseline (speedup 1.0000x reference)
import functools

import jax
import jax.numpy as jnp
from jax import lax
from jax.experimental import pallas as pl
from jax.experimental.pallas import tpu as pltpu
from jax.experimental.pallas import tpu_sc as plsc

BATCH = 16384
EMBED_DIM = 64
_NC = 2
_NS = 16
_NW = _NC * _NS
_BPW = BATCH // _NW
_CHUNK = 128
_NCHUNK = _BPW // _CHUNK


def _cf_body(uid_hbm, iid_hbm, ut_hbm, it_hbm, out_hbm,
             uidx_v, iidx_v, urows_v, irows_v, out_v, sem):
    wid = lax.axis_index("s") * _NC + lax.axis_index("c")
    base = wid * _BPW

    pltpu.sync_copy(uid_hbm.at[pl.ds(base, _BPW)], uidx_v)
    pltpu.sync_copy(iid_hbm.at[pl.ds(base, _BPW)], iidx_v)

    copies = []
    for j in range(_NCHUNK):
        s = pl.ds(j * _CHUNK, _CHUNK)
        copies.append(pltpu.async_copy(
            ut_hbm.at[uidx_v.at[s]], urows_v.at[s, :], sem))
        copies.append(pltpu.async_copy(
            it_hbm.at[iidx_v.at[s]], irows_v.at[s, :], sem))
    for cp in copies:
        cp.wait()

    lanes = lax.iota(jnp.int32, 16)

    def group(g, _):
        base_r = g * 16
        acc = jnp.zeros((16,), jnp.float32)
        for r in range(16):
            row = base_r + r
            p = urows_v[row, pl.ds(0, 16)] * irows_v[row, pl.ds(0, 16)]
            for k in range(1, EMBED_DIM // 16):
                p += urows_v[row, pl.ds(k * 16, 16)] * irows_v[row, pl.ds(k * 16, 16)]
            acc = jnp.where(lanes == r, jnp.sum(p), acc)
        out_v[pl.ds(base_r, 16)] = acc
        return 0

    lax.fori_loop(0, _BPW // 16, group, 0)

    pltpu.sync_copy(out_v, out_hbm.at[pl.ds(base, _BPW)])


@jax.jit
def _cf_kernel(user_id, item_id, user_table, item_table):
    mesh = plsc.VectorSubcoreMesh(core_axis_name="c", subcore_axis_name="s")
    f = pl.kernel(
        _cf_body,
        out_type=jax.ShapeDtypeStruct((BATCH,), jnp.float32),
        mesh=mesh,
        scratch_types=[
            pltpu.VMEM((_BPW,), jnp.int32),
            pltpu.VMEM((_BPW,), jnp.int32),
            pltpu.VMEM((_BPW, EMBED_DIM), jnp.float32),
            pltpu.VMEM((_BPW, EMBED_DIM), jnp.float32),
            pltpu.VMEM((_BPW,), jnp.float32),
            pltpu.SemaphoreType.DMA,
        ],
        compiler_params=pltpu.CompilerParams(
            needs_layout_passes=False, use_tc_tiling_on_sc=False),
    )
    return f(user_id, item_id, user_table, item_table)


def kernel(user_id, item_id, user_table, item_table):
    out = _cf_kernel(user_id, item_id, user_table, item_table)
    return out.reshape(BATCH, 1)
